# baseline (speedup 1.0000x reference)
import jax
import jax.numpy as jnp
from jax import lax
from jax.experimental import pallas as pl
from jax.experimental.pallas import tpu as pltpu
from jax.experimental.pallas import tpu_sc as plsc

B = 4096
L = 50
V = 1000000
D = 64
NC = 2
NS = 16
NW = NC * NS

VB = 384
TFULL = V // VB
TTAIL = V - TFULL * VB
TROUNDS = -(-TFULL // NW)
OUTW = VB * D

ROWS_TOTAL = 2 * B
ROWS_PER_W = ROWS_TOTAL // NW
SEGS_PER_CHUNK = 2
CHUNK = SEGS_PER_CHUNK * L
NCHUNK = ROWS_PER_W // SEGS_PER_CHUNK
NBUF = 2


def _worker_id():
    return lax.axis_index("s") * NC + lax.axis_index("c")


def _transpose_block(src, dst, ncols):
    lanes = lax.iota(jnp.int32, 16)

    def col_body(v, carry):
        cols = jnp.full((16,), v, jnp.int32)
        for j in range(D // 16):
            dst[pl.ds(v * D + 16 * j, 16)] = plsc.load_gather(
                src, [lanes + 16 * j, cols]
            )
        return carry

    lax.fori_loop(0, ncols, col_body, 0, unroll=4)


def _sc_transpose_body(tableT_hbm, tail_hbm, flat_hbm, in0, in1, out0, out1,
                       tbuf, insems, outsems):
    w = _worker_id()
    bufs = ((in0, out0), (in1, out1))

    def issue_in(c, half):
        return pltpu.async_copy(
            tableT_hbm.at[:, pl.ds(c * VB, VB)], bufs[half][0].at[:, pl.ds(0, VB)],
            insems.at[half],
        )

    def issue_out(c, half):
        return pltpu.async_copy(
            bufs[half][1], flat_hbm.at[pl.ds(c * OUTW, OUTW)],
            outsems.at[half],
        )

    for k in (0, 1):
        issue_in(w + NW * k, k)

    def round_body(k2, carry):
        for half in (0, 1):
            ib, ob = bufs[half]
            k = 2 * k2 + half
            c = w + NW * k

            @pl.when(c < TFULL)
            def _():
                pltpu.make_async_copy(
                    tableT_hbm.at[:, pl.ds(c * VB, VB)], ib.at[:, pl.ds(0, VB)], insems.at[half]
                ).wait()

                @pl.when(k >= 2)
                def _():
                    pltpu.make_async_copy(
                        ob,
                        flat_hbm.at[pl.ds((c - 2 * NW) * OUTW, OUTW)],
                        outsems.at[half],
                    ).wait()

                _transpose_block(ib, ob, VB)

                @pl.when(c + 2 * NW < TFULL)
                def _():
                    issue_in(c + 2 * NW, half)

                issue_out(c, half)

        return carry

    lax.fori_loop(0, TROUNDS // 2, round_body, 0)

    for k in (TROUNDS - 2, TROUNDS - 1):
        c = w + NW * k

        @pl.when(c < TFULL)
        def _():
            pltpu.make_async_copy(
                bufs[k % 2][1],
                flat_hbm.at[pl.ds(c * OUTW, OUTW)],
                outsems.at[k % 2],
            ).wait()

    @pl.when(w == NW - 1)
    def _():
        pltpu.sync_copy(tail_hbm, tbuf)
        pltpu.sync_copy(tbuf, flat_hbm.at[pl.ds(TFULL * VB * D, TTAIL * D)])


def _sc_pool_body(idx_hbm, table_hbm, out_hbm, idx_v, rows_v, stage_v, sems):
    wid = _worker_id()

    pltpu.sync_copy(idx_hbm.at[wid], idx_v)

    def start_gather(ch, buf):
        return pltpu.async_copy(
            table_hbm.at[idx_v.at[ch]], rows_v.at[buf], sems.at[buf]
        )

    for b in range(NBUF):
        start_gather(b, b)

    def chunk_body(ch, carry):
        buf = lax.rem(ch, NBUF)
        pltpu.make_async_copy(
            table_hbm.at[idx_v.at[ch]], rows_v.at[buf], sems.at[buf]
        ).wait()

        for seg in range(SEGS_PER_CHUNK):
            def row_body(r, accs):
                base = seg * L + r
                return tuple(
                    accs[k] + rows_v[buf, base, pl.ds(k * 16, 16)]
                    for k in range(D // 16)
                )

            zeros = tuple(
                jnp.zeros((16,), jnp.float32) for _ in range(D // 16)
            )
            accs = lax.fori_loop(0, L, row_body, zeros, unroll=5)
            for k in range(D // 16):
                stage_v[SEGS_PER_CHUNK * ch + seg, pl.ds(k * 16, 16)] = (
                    accs[k] * (1.0 / L)
                )

        @pl.when(ch + NBUF < NCHUNK)
        def _():
            start_gather(ch + NBUF, buf)

        return carry

    lax.fori_loop(0, NCHUNK, chunk_body, 0)

    pltpu.sync_copy(stage_v, out_hbm.at[pl.ds(wid * ROWS_PER_W, ROWS_PER_W)])


@jax.jit
def _sc_gather_mean(idx, table):
    mesh = plsc.VectorSubcoreMesh(core_axis_name="c", subcore_axis_name="s")
    flat = pl.kernel(
        _sc_transpose_body,
        out_type=jax.ShapeDtypeStruct((V * D,), jnp.float32),
        mesh=mesh,
        scratch_types=[
            pltpu.VMEM((D, VB + 1), jnp.float32),
            pltpu.VMEM((D, VB + 1), jnp.float32),
            pltpu.VMEM((OUTW,), jnp.float32),
            pltpu.VMEM((OUTW,), jnp.float32),
            pltpu.VMEM((TTAIL * D,), jnp.float32),
            pltpu.SemaphoreType.DMA((2,)),
            pltpu.SemaphoreType.DMA((2,)),
        ],
        compiler_params=pltpu.CompilerParams(needs_layout_passes=False),
    )(table.T, table[TFULL * VB:, :].reshape(-1))

    return pl.kernel(
        _sc_pool_body,
        out_type=jax.ShapeDtypeStruct((ROWS_TOTAL, D), jnp.float32),
        mesh=mesh,
        scratch_types=[
            pltpu.VMEM((NCHUNK, CHUNK), jnp.int32),
            pltpu.VMEM((NBUF, CHUNK, D), jnp.float32),
            pltpu.VMEM((ROWS_PER_W, D), jnp.float32),
            pltpu.SemaphoreType.DMA((NBUF,)),
        ],
        compiler_params=pltpu.CompilerParams(use_tc_tiling_on_sc=False),
    )(idx, flat.reshape(V, D))


def _tc_mlp_body(u_ref, v_ref, w1t_ref, b1_ref, w2t_ref, b2_ref, out_ref):
    u = u_ref[...]
    v = v_ref[...]
    combined = jnp.concatenate([u, v, jnp.abs(u - v), u * v], axis=1)
    h = jnp.dot(combined, w1t_ref[...], preferred_element_type=jnp.float32)
    h = jnp.maximum(h + b1_ref[...], 0.0)
    logits = jnp.dot(h, w2t_ref[...], preferred_element_type=jnp.float32)
    logits = logits + b2_ref[...]
    m = jnp.max(logits, axis=1, keepdims=True)
    e = jnp.exp(logits - m)
    out_ref[...] = e / jnp.sum(e, axis=1, keepdims=True)


@jax.jit
def _tc_mlp(u, v, w1t, b1, w2t, b2):
    return pl.pallas_call(
        _tc_mlp_body,
        out_shape=jax.ShapeDtypeStruct((B, w2t.shape[1]), jnp.float32),
    )(u, v, w1t, b1, w2t, b2)


@jax.jit
def kernel(sentence1, sentence2, table, W1, b1, W2, b2):
    idx = jnp.concatenate(
        [sentence1.reshape(-1), sentence2.reshape(-1)]
    ).reshape(NW, NCHUNK, CHUNK)
    uv = _sc_gather_mean(idx, table)
    u = uv[:B]
    v = uv[B:]
    nl = W2.shape[0]
    out = _tc_mlp(
        u, v, W1.T, b1.reshape(1, -1), W2.T, b2.reshape(1, -1)
    )
    return out[:, :nl]


# speedup vs baseline: 1.7828x; 1.7828x over previous
import functools

import jax
import jax.numpy as jnp
import numpy as np
from jax import lax
from jax.experimental import pallas as pl
from jax.experimental.pallas import tpu as pltpu
from jax.experimental.pallas import tpu_sc as plsc

B = 4096
L = 50
V = 1000000
D = 64
NC = 2
NS = 16
NW = NC * NS

VB = 384
TFULL = V // VB
TTAIL = V - TFULL * VB
TROUNDS = -(-TFULL // NW)
OUTW = VB * D

ROWS_TOTAL = 2 * B
ROWS_PER_W = ROWS_TOTAL // NW
SEGS_PER_CHUNK = 2
CHUNK = SEGS_PER_CHUNK * L
NCHUNK = ROWS_PER_W // SEGS_PER_CHUNK
NBUF = 2


def _worker_id():
    return lax.axis_index("s") * NC + lax.axis_index("c")


def _transpose_block(src, dst, ncols):
    lanes = lax.iota(jnp.int32, 16)

    def col_body(v, carry):
        cols = jnp.full((16,), v, jnp.int32)
        for j in range(D // 16):
            dst[pl.ds(v * D + 16 * j, 16)] = plsc.load_gather(
                src, [lanes + 16 * j, cols]
            )
        return carry

    lax.fori_loop(0, ncols, col_body, 0, unroll=4)


def _sc_transpose_body(tableT_hbm, tail_hbm, flat_hbm, in0, in1, out0, out1,
                       tbuf, insems, outsems):
    w = _worker_id()
    bufs = ((in0, out0), (in1, out1))

    def issue_in(c, half):
        return pltpu.async_copy(
            tableT_hbm.at[:, pl.ds(c * VB, VB)], bufs[half][0].at[:, pl.ds(0, VB)],
            insems.at[half],
        )

    def issue_out(c, half):
        return pltpu.async_copy(
            bufs[half][1], flat_hbm.at[pl.ds(c * OUTW, OUTW)],
            outsems.at[half],
        )

    for k in (0, 1):
        issue_in(w + NW * k, k)

    def round_body(k2, carry):
        for half in (0, 1):
            ib, ob = bufs[half]
            k = 2 * k2 + half
            c = w + NW * k

            @pl.when(c < TFULL)
            def _():
                pltpu.make_async_copy(
                    tableT_hbm.at[:, pl.ds(c * VB, VB)], ib.at[:, pl.ds(0, VB)], insems.at[half]
                ).wait()

                @pl.when(k >= 2)
                def _():
                    pltpu.make_async_copy(
                        ob,
                        flat_hbm.at[pl.ds((c - 2 * NW) * OUTW, OUTW)],
                        outsems.at[half],
                    ).wait()

                _transpose_block(ib, ob, VB)

                @pl.when(c + 2 * NW < TFULL)
                def _():
                    issue_in(c + 2 * NW, half)

                issue_out(c, half)

        return carry

    lax.fori_loop(0, TROUNDS // 2, round_body, 0)

    for k in (TROUNDS - 2, TROUNDS - 1):
        c = w + NW * k

        @pl.when(c < TFULL)
        def _():
            pltpu.make_async_copy(
                bufs[k % 2][1],
                flat_hbm.at[pl.ds(c * OUTW, OUTW)],
                outsems.at[k % 2],
            ).wait()

    @pl.when(w == NW - 1)
    def _():
        pltpu.sync_copy(tail_hbm, tbuf)
        pltpu.sync_copy(tbuf, flat_hbm.at[pl.ds(TFULL * VB * D, TTAIL * D)])


def _sc_pool_body(idx_hbm, table_hbm, out_hbm, idx_v, rows_v, stage_v, sems):
    wid = _worker_id()

    pltpu.sync_copy(idx_hbm.at[wid], idx_v)

    def start_gather(ch, buf):
        return pltpu.async_copy(
            table_hbm.at[idx_v.at[ch]], rows_v.at[buf], sems.at[buf]
        )

    for b in range(NBUF):
        start_gather(b, b)

    def chunk_body(ch, carry):
        buf = lax.rem(ch, NBUF)
        pltpu.make_async_copy(
            table_hbm.at[idx_v.at[ch]], rows_v.at[buf], sems.at[buf]
        ).wait()

        for seg in range(SEGS_PER_CHUNK):
            def row_body(r, accs):
                base = seg * L + r
                out = list(accs)
                for k2 in range(D // 32):
                    a, b = plsc.unpack(
                        rows_v[buf, base, pl.ds(32 * k2, 32)],
                        format=plsc.PackFormat.INTERLEAVED,
                    )
                    out[2 * k2] = out[2 * k2] + a
                    out[2 * k2 + 1] = out[2 * k2 + 1] + b
                return tuple(out)

            zeros = tuple(
                jnp.zeros((16,), jnp.float32) for _ in range(D // 16)
            )
            accs = lax.fori_loop(0, L, row_body, zeros, unroll=5)
            for k in range(D // 16):
                stage_v[SEGS_PER_CHUNK * ch + seg, pl.ds(k * 16, 16)] = (
                    accs[k] * (1.0 / L)
                )

        @pl.when(ch + NBUF < NCHUNK)
        def _():
            start_gather(ch + NBUF, buf)

        return carry

    lax.fori_loop(0, NCHUNK, chunk_body, 0)

    pltpu.sync_copy(stage_v, out_hbm.at[pl.ds(wid * ROWS_PER_W, ROWS_PER_W)])


@jax.jit
def _sc_gather_mean(idx, table):
    mesh = plsc.VectorSubcoreMesh(core_axis_name="c", subcore_axis_name="s")
    return pl.kernel(
        _sc_pool_body,
        out_type=jax.ShapeDtypeStruct((ROWS_TOTAL, D), jnp.float32),
        mesh=mesh,
        scratch_types=[
            pltpu.VMEM((NCHUNK, CHUNK), jnp.int32),
            pltpu.VMEM((NBUF, CHUNK, D), jnp.bfloat16),
            pltpu.VMEM((ROWS_PER_W, D), jnp.float32),
            pltpu.SemaphoreType.DMA((NBUF,)),
        ],
        compiler_params=pltpu.CompilerParams(
            use_tc_tiling_on_sc=False, needs_layout_passes=False
        ),
    )(idx, table.astype(jnp.bfloat16))


def _tc_mlp_body(u_ref, v_ref, w1t_ref, b1_ref, w2t_ref, b2_ref, out_ref):
    u = u_ref[...]
    v = v_ref[...]
    combined = jnp.concatenate([u, v, jnp.abs(u - v), u * v], axis=1)
    h = jnp.dot(combined, w1t_ref[...], preferred_element_type=jnp.float32)
    h = jnp.maximum(h + b1_ref[...], 0.0)
    logits = jnp.dot(h, w2t_ref[...], preferred_element_type=jnp.float32)
    logits = logits + b2_ref[...]
    m = jnp.max(logits, axis=1, keepdims=True)
    e = jnp.exp(logits - m)
    out_ref[...] = e / jnp.sum(e, axis=1, keepdims=True)


@jax.jit
def _tc_mlp(u, v, w1t, b1, w2t, b2):
    return pl.pallas_call(
        _tc_mlp_body,
        out_shape=jax.ShapeDtypeStruct((B, w2t.shape[1]), jnp.float32),
    )(u, v, w1t, b1, w2t, b2)


@jax.jit
def kernel(sentence1, sentence2, table, W1, b1, W2, b2):
    idx = jnp.concatenate(
        [sentence1.reshape(-1), sentence2.reshape(-1)]
    ).reshape(NW, NCHUNK, CHUNK)
    uv = _sc_gather_mean(idx, table)
    u = uv[:B]
    v = uv[B:]
    nl = W2.shape[0]
    perm = np.concatenate(
        [blk + np.concatenate([np.arange(0, 32, 2), np.arange(1, 32, 2)])
         for blk in range(0, D, 32)]
    )
    perm_full = np.concatenate([g * D + perm for g in range(4)])
    out = _tc_mlp(
        u, v, W1.T[perm_full], b1.reshape(1, -1), W2.T, b2.reshape(1, -1)
    )
    return out[:, :nl]


# speedup vs baseline: 2.2940x; 1.2867x over previous
import functools

import jax
import jax.numpy as jnp
import numpy as np
from jax import lax
from jax.experimental import pallas as pl
from jax.experimental.pallas import tpu as pltpu
from jax.experimental.pallas import tpu_sc as plsc

B = 4096
L = 50
V = 1000000
D = 64
NC = 2
NS = 16
NW = NC * NS

VB = 384
TFULL = V // VB
TTAIL = V - TFULL * VB
TROUNDS = -(-TFULL // NW)
OUTW = VB * D

ROWS_TOTAL = 2 * B
ROWS_PER_W = ROWS_TOTAL // NW
SEGS_PER_CHUNK = 2
CHUNK = SEGS_PER_CHUNK * L
NCHUNK = ROWS_PER_W // SEGS_PER_CHUNK
NBUF = 2


def _worker_id():
    return lax.axis_index("s") * NC + lax.axis_index("c")


def _transpose_block(src, dst, ncols):
    lanes = lax.iota(jnp.int32, 16)

    def col_body(v, carry):
        cols = jnp.full((16,), v, jnp.int32)
        for j in range(D // 16):
            dst[pl.ds(v * D + 16 * j, 16)] = plsc.load_gather(
                src, [lanes + 16 * j, cols]
            )
        return carry

    lax.fori_loop(0, ncols, col_body, 0, unroll=4)


def _sc_transpose_body(tableT_hbm, tail_hbm, flat_hbm, in0, in1, out0, out1,
                       tbuf, insems, outsems):
    w = _worker_id()
    bufs = ((in0, out0), (in1, out1))

    def issue_in(c, half):
        return pltpu.async_copy(
            tableT_hbm.at[:, pl.ds(c * VB, VB)], bufs[half][0].at[:, pl.ds(0, VB)],
            insems.at[half],
        )

    def issue_out(c, half):
        return pltpu.async_copy(
            bufs[half][1], flat_hbm.at[pl.ds(c * OUTW, OUTW)],
            outsems.at[half],
        )

    for k in (0, 1):
        issue_in(w + NW * k, k)

    def round_body(k2, carry):
        for half in (0, 1):
            ib, ob = bufs[half]
            k = 2 * k2 + half
            c = w + NW * k

            @pl.when(c < TFULL)
            def _():
                pltpu.make_async_copy(
                    tableT_hbm.at[:, pl.ds(c * VB, VB)], ib.at[:, pl.ds(0, VB)], insems.at[half]
                ).wait()

                @pl.when(k >= 2)
                def _():
                    pltpu.make_async_copy(
                        ob,
                        flat_hbm.at[pl.ds((c - 2 * NW) * OUTW, OUTW)],
                        outsems.at[half],
                    ).wait()

                _transpose_block(ib, ob, VB)

                @pl.when(c + 2 * NW < TFULL)
                def _():
                    issue_in(c + 2 * NW, half)

                issue_out(c, half)

        return carry

    lax.fori_loop(0, TROUNDS // 2, round_body, 0)

    for k in (TROUNDS - 2, TROUNDS - 1):
        c = w + NW * k

        @pl.when(c < TFULL)
        def _():
            pltpu.make_async_copy(
                bufs[k % 2][1],
                flat_hbm.at[pl.ds(c * OUTW, OUTW)],
                outsems.at[k % 2],
            ).wait()

    @pl.when(w == NW - 1)
    def _():
        pltpu.sync_copy(tail_hbm, tbuf)
        pltpu.sync_copy(tbuf, flat_hbm.at[pl.ds(TFULL * VB * D, TTAIL * D)])


def _sc_pool_body(idx_hbm, table_hbm, out_hbm, idx_v, rows_v, stage_v, sems):
    wid = _worker_id()

    pltpu.sync_copy(idx_hbm.at[wid], idx_v)

    def start_gather(ch, buf):
        return pltpu.async_copy(
            table_hbm.at[idx_v.at[ch]], rows_v.at[buf], sems.at[buf]
        )

    for b in range(NBUF):
        start_gather(b, b)

    def chunk_body(ch, carry):
        buf = lax.rem(ch, NBUF)
        pltpu.make_async_copy(
            table_hbm.at[idx_v.at[ch]], rows_v.at[buf], sems.at[buf]
        ).wait()

        for seg in range(SEGS_PER_CHUNK):
            def row_body(r, accs):
                base = seg * L + r
                return tuple(
                    accs[k] + rows_v[buf, base, pl.ds(k * 16, 16)]
                    for k in range(D // 16)
                )

            zeros = tuple(
                jnp.zeros((16,), jnp.float32) for _ in range(D // 16)
            )
            accs = lax.fori_loop(0, L, row_body, zeros, unroll=5)
            for k in range(D // 16):
                stage_v[SEGS_PER_CHUNK * ch + seg, pl.ds(k * 16, 16)] = (
                    accs[k] * (1.0 / L)
                )

        @pl.when(ch + NBUF < NCHUNK)
        def _():
            start_gather(ch + NBUF, buf)

        return carry

    lax.fori_loop(0, NCHUNK, chunk_body, 0)

    pltpu.sync_copy(stage_v, out_hbm.at[pl.ds(wid * ROWS_PER_W, ROWS_PER_W)])


@jax.jit
def _sc_gather_mean(idx, table):
    mesh = plsc.VectorSubcoreMesh(core_axis_name="c", subcore_axis_name="s")
    return pl.kernel(
        _sc_pool_body,
        out_type=jax.ShapeDtypeStruct((ROWS_TOTAL, D), jnp.float32),
        mesh=mesh,
        scratch_types=[
            pltpu.VMEM((NCHUNK, CHUNK), jnp.int32),
            pltpu.VMEM((NBUF, CHUNK, D), jnp.float32),
            pltpu.VMEM((ROWS_PER_W, D), jnp.float32),
            pltpu.SemaphoreType.DMA((NBUF,)),
        ],
        compiler_params=pltpu.CompilerParams(use_tc_tiling_on_sc=False),
    )(idx, table)


def _tc_mlp_body(u_ref, v_ref, w1t_ref, b1_ref, w2t_ref, b2_ref, out_ref):
    u = u_ref[...]
    v = v_ref[...]
    combined = jnp.concatenate([u, v, jnp.abs(u - v), u * v], axis=1)
    h = jnp.dot(combined, w1t_ref[...], preferred_element_type=jnp.float32)
    h = jnp.maximum(h + b1_ref[...], 0.0)
    logits = jnp.dot(h, w2t_ref[...], preferred_element_type=jnp.float32)
    logits = logits + b2_ref[...]
    m = jnp.max(logits, axis=1, keepdims=True)
    e = jnp.exp(logits - m)
    out_ref[...] = e / jnp.sum(e, axis=1, keepdims=True)


@jax.jit
def _tc_mlp(u, v, w1t, b1, w2t, b2):
    return pl.pallas_call(
        _tc_mlp_body,
        out_shape=jax.ShapeDtypeStruct((B, w2t.shape[1]), jnp.float32),
    )(u, v, w1t, b1, w2t, b2)


@jax.jit
def kernel(sentence1, sentence2, table, W1, b1, W2, b2):
    idx = jnp.concatenate(
        [sentence1.reshape(-1), sentence2.reshape(-1)]
    ).reshape(NW, NCHUNK, CHUNK)
    uv = _sc_gather_mean(idx, table)
    u = uv[:B]
    v = uv[B:]
    nl = W2.shape[0]
    out = _tc_mlp(
        u, v, W1.T, b1.reshape(1, -1), W2.T, b2.reshape(1, -1)
    )
    return out[:, :nl]


# speedup vs baseline: 2.2950x; 1.0005x over previous
import jax
import jax.numpy as jnp
from jax import lax
from jax.experimental import pallas as pl
from jax.experimental.pallas import tpu as pltpu
from jax.experimental.pallas import tpu_sc as plsc

B = 4096
L = 50
V = 1000000
D = 64
NC = 2
NS = 16
NW = NC * NS

ROWS_TOTAL = 2 * B
ROWS_PER_W = ROWS_TOTAL // NW
SEGS_PER_CHUNK = 2
CHUNK = SEGS_PER_CHUNK * L
NCHUNK = ROWS_PER_W // SEGS_PER_CHUNK
NBUF = 2


def _worker_id():
    return lax.axis_index("s") * NC + lax.axis_index("c")


def _sc_pool_body(idx_hbm, table_hbm, out_hbm, idx_v, rows_v, stage_v, sems):
    wid = _worker_id()

    pltpu.sync_copy(idx_hbm.at[wid], idx_v)

    def start_gather(ch, buf):
        return pltpu.async_copy(
            table_hbm.at[idx_v.at[ch]], rows_v.at[buf], sems.at[buf]
        )

    for b in range(NBUF):
        start_gather(b, b)

    def chunk_body(ch, carry):
        buf = lax.rem(ch, NBUF)
        pltpu.make_async_copy(
            table_hbm.at[idx_v.at[ch]], rows_v.at[buf], sems.at[buf]
        ).wait()

        for seg in range(SEGS_PER_CHUNK):
            def row_body(r, accs):
                base = seg * L + r
                return tuple(
                    accs[k] + rows_v[buf, base, pl.ds(k * 16, 16)]
                    for k in range(D // 16)
                )

            zeros = tuple(
                jnp.zeros((16,), jnp.float32) for _ in range(D // 16)
            )
            accs = lax.fori_loop(0, L, row_body, zeros, unroll=5)
            for k in range(D // 16):
                stage_v[SEGS_PER_CHUNK * ch + seg, pl.ds(k * 16, 16)] = (
                    accs[k] * (1.0 / L)
                )

        @pl.when(ch + NBUF < NCHUNK)
        def _():
            start_gather(ch + NBUF, buf)

        return carry

    lax.fori_loop(0, NCHUNK, chunk_body, 0)

    pltpu.sync_copy(stage_v, out_hbm.at[pl.ds(wid * ROWS_PER_W, ROWS_PER_W)])


@jax.jit
def _sc_gather_mean(idx, table):
    mesh = plsc.VectorSubcoreMesh(core_axis_name="c", subcore_axis_name="s")
    return pl.kernel(
        _sc_pool_body,
        out_type=jax.ShapeDtypeStruct((ROWS_TOTAL, D), jnp.float32),
        mesh=mesh,
        scratch_types=[
            pltpu.VMEM((NCHUNK, CHUNK), jnp.int32),
            pltpu.VMEM((NBUF, CHUNK, D), jnp.float32),
            pltpu.VMEM((ROWS_PER_W, D), jnp.float32),
            pltpu.SemaphoreType.DMA((NBUF,)),
        ],
        compiler_params=pltpu.CompilerParams(use_tc_tiling_on_sc=False),
    )(idx, table)


def _tc_mlp_body(u_ref, v_ref, w1t_ref, b1_ref, w2t_ref, b2_ref, out_ref):
    u = u_ref[...]
    v = v_ref[...]
    combined = jnp.concatenate([u, v, jnp.abs(u - v), u * v], axis=1)
    h = jnp.dot(combined, w1t_ref[...], preferred_element_type=jnp.float32)
    h = jnp.maximum(h + b1_ref[...], 0.0)
    logits = jnp.dot(h, w2t_ref[...], preferred_element_type=jnp.float32)
    logits = logits + b2_ref[...]
    m = jnp.max(logits, axis=1, keepdims=True)
    e = jnp.exp(logits - m)
    out_ref[...] = e / jnp.sum(e, axis=1, keepdims=True)


@jax.jit
def _tc_mlp(u, v, w1t, b1, w2t, b2):
    return pl.pallas_call(
        _tc_mlp_body,
        out_shape=jax.ShapeDtypeStruct((B, w2t.shape[1]), jnp.float32),
    )(u, v, w1t, b1, w2t, b2)


@jax.jit
def kernel(sentence1, sentence2, table, W1, b1, W2, b2):
    idx = jnp.concatenate(
        [sentence1.reshape(-1), sentence2.reshape(-1)]
    ).reshape(NW, NCHUNK, CHUNK)
    uv = _sc_gather_mean(idx, table)
    u = uv[:B]
    v = uv[B:]
    nl = W2.shape[0]
    out = _tc_mlp(
        u, v, W1.T, b1.reshape(1, -1), W2.T, b2.reshape(1, -1)
    )
    return out[:, :nl]
